# baseline (speedup 1.0000x reference)
import jax
import jax.numpy as jnp
from jax import lax
from jax.experimental import pallas as pl
from jax.experimental.pallas import tpu as pltpu
from jax.experimental.pallas import tpu_sc as plsc

MINPOS = -32
NBIN = 65
D = 128
L = 512
N_ROWS = L * L

_info = plsc.get_sparse_core_info()
NC, NS = _info.num_cores, _info.num_subcores
NW = NC * NS
I_PER_W = L // NW
CHUNK = 128
UNITS = I_PER_W * (L // CHUNK)


def _body(table_hbm, idx_hbm, out_hbm, idx_v, bins_v, buf_v, sem):
    wid = lax.axis_index("s") * NC + lax.axis_index("c")
    ibase = wid * I_PER_W

    pltpu.sync_copy(idx_hbm, idx_v.at[pl.ds(0, L)])

    def unit(u, carry):
        i = ibase + u // (L // CHUNK)
        j0 = (u % (L // CHUNK)) * CHUNK
        cvec = idx_v[pl.ds(i, 16)]
        ivec = jnp.full((16,), cvec[0], jnp.int32)
        for c in range(CHUNK // 16):
            jvec = idx_v[pl.ds(j0 + c * 16, 16)]
            b = jnp.minimum(jnp.maximum(jvec - ivec + 32, 0), NBIN - 1)
            bins_v[pl.ds(c * 16, 16)] = b
        pltpu.async_copy(table_hbm.at[bins_v], buf_v, sem).wait()
        pltpu.sync_copy(buf_v, out_hbm.at[pl.ds(i * L + j0, CHUNK)])
        return carry

    lax.fori_loop(0, UNITS, unit, 0)


def kernel(idx, emb_table):
    idx_flat = idx.reshape(L).astype(jnp.int32)
    mesh = plsc.VectorSubcoreMesh(core_axis_name="c", subcore_axis_name="s")
    out = pl.kernel(
        _body,
        mesh=mesh,
        out_type=jax.ShapeDtypeStruct((N_ROWS, D), jnp.float32),
        scratch_types=[
            pltpu.VMEM((L + 16,), jnp.int32),
            pltpu.VMEM((CHUNK,), jnp.int32),
            pltpu.VMEM((CHUNK, D), jnp.float32),
            pltpu.SemaphoreType.DMA,
        ],
    )(emb_table, idx_flat)
    return out.reshape(1, L, L, D)
